# baseline (speedup 1.0000x reference)
import functools

import jax
import jax.numpy as jnp
from jax import lax
from jax.experimental import pallas as pl
from jax.experimental.pallas import tpu as pltpu
from jax.experimental.pallas import tpu_sc as plsc

N = 1000000
DIM = 16
B = 16384

_NC = 2
_NS = 16
_NW = _NC * _NS
_BPW = B // _NW
_CHUNK = 128
_NCHUNK = _BPW // _CHUNK
_NGROUP = _BPW // 16
_SROW = _BPW // 128


def _sc_scores(left, right, embt, bias):
    mesh = plsc.VectorSubcoreMesh(core_axis_name="c", subcore_axis_name="s")

    @functools.partial(
        pl.kernel,
        out_type=jax.ShapeDtypeStruct((128, 128), jnp.float32),
        mesh=mesh,
        scratch_types=[
            pltpu.VMEM((_NCHUNK, _CHUNK), jnp.int32),
            pltpu.VMEM((_NCHUNK, _CHUNK), jnp.int32),
            pltpu.VMEM((DIM, _BPW), jnp.float32),
            pltpu.VMEM((DIM, _BPW), jnp.float32),
            pltpu.VMEM((_BPW,), jnp.float32),
            pltpu.VMEM((_SROW, 128), jnp.float32),
            pltpu.SemaphoreType.DMA,
        ],
        compiler_params=pltpu.CompilerParams(use_tc_tiling_on_sc=False),
    )
    def body(left_hbm, right_hbm, embt_hbm, bias_hbm, score_hbm,
             lidx, ridx, lcols, rcols, bvals, score_v, sem):
        wid = lax.axis_index("s") * _NC + lax.axis_index("c")
        base = wid * _BPW

        for c in range(_NCHUNK):
            pltpu.sync_copy(left_hbm.at[pl.ds(base + c * _CHUNK, _CHUNK)],
                            lidx.at[c])
            pltpu.sync_copy(right_hbm.at[pl.ds(base + c * _CHUNK, _CHUNK)],
                            ridx.at[c])

        handles = []
        for c in range(_NCHUNK):
            sl = pl.ds(c * _CHUNK, _CHUNK)
            handles.append(pltpu.async_copy(bias_hbm.at[ridx.at[c]],
                                            bvals.at[sl], sem))
            for j in range(DIM):
                handles.append(pltpu.async_copy(
                    embt_hbm.at[j].at[lidx.at[c]], lcols.at[j, sl], sem))
                handles.append(pltpu.async_copy(
                    embt_hbm.at[j].at[ridx.at[c]], rcols.at[j, sl], sem))
        for h in handles:
            h.wait()

        for g in range(_NGROUP):
            sl = pl.ds(g * 16, 16)
            acc = bvals[sl]
            for j in range(DIM):
                acc = acc + lcols[j, sl] * rcols[j, sl]
            score_v[g // 8, pl.ds((g % 8) * 16, 16)] = acc

        pltpu.sync_copy(score_v, score_hbm.at[pl.ds(wid * _SROW, _SROW)])

    return body(left, right, embt, bias)


def _tc_loss_kernel(score_ref, y_ref, out_ref):
    s = score_ref[...]
    y = y_ref[...]
    prob = jax.nn.sigmoid(s)
    prob = jnp.clip(prob, 1e-05, 1 - 1e-05)
    out_ref[0, 0] = -jnp.sum(y * jnp.log(prob) + (1 - y) * jnp.log(1 - prob))


def _tc_loss(score, y):
    out = pl.pallas_call(
        _tc_loss_kernel,
        out_shape=jax.ShapeDtypeStruct((1, 1), jnp.float32),
        out_specs=pl.BlockSpec(memory_space=pltpu.SMEM),
    )(score, y.reshape(128, 128))
    return out[0, 0]


def kernel(left, right, y, emb, bias):
    score = _sc_scores(left.astype(jnp.int32), right.astype(jnp.int32),
                       emb.T, bias)
    return _tc_loss(score, y)


# speedup vs baseline: 2.7484x; 2.7484x over previous
import functools

import jax
import jax.numpy as jnp
from jax import lax
from jax.experimental import pallas as pl
from jax.experimental.pallas import tpu as pltpu
from jax.experimental.pallas import tpu_sc as plsc

N = 1000000
DIM = 16
B = 16384

_NC = 2
_NS = 16
_NW = _NC * _NS

_TCOLS = N // 128
_SLABS_PER_TILE = _TCOLS // _NW
_EXTRA = _TCOLS - _SLABS_PER_TILE * _NW
_TAIL0 = _TCOLS * 128
_NBUF = 4
_GROUPS = _SLABS_PER_TILE // _NBUF

_BPW = B // _NW
_CHUNK = 128
_NCHUNK = _BPW // _CHUNK
_NGROUP = _BPW // 16


def _sc_convert(embt, tail):
    mesh = plsc.VectorSubcoreMesh(core_axis_name="c", subcore_axis_name="s")

    @functools.partial(
        pl.kernel,
        out_type=jax.ShapeDtypeStruct((N * DIM,), jnp.float32),
        mesh=mesh,
        scratch_types=[
            pltpu.VMEM((DIM, 128), jnp.float32),
            pltpu.VMEM((DIM, 128), jnp.float32),
            pltpu.VMEM((DIM, 128), jnp.float32),
            pltpu.VMEM((DIM, 128), jnp.float32),
            pltpu.VMEM((2048,), jnp.float32),
            pltpu.VMEM((2048,), jnp.float32),
            pltpu.VMEM((2048,), jnp.float32),
            pltpu.VMEM((2048,), jnp.float32),
            pltpu.VMEM((1024,), jnp.float32),
            pltpu.SemaphoreType.DMA,
            pltpu.SemaphoreType.DMA,
            pltpu.SemaphoreType.DMA,
            pltpu.SemaphoreType.DMA,
            pltpu.SemaphoreType.DMA,
            pltpu.SemaphoreType.DMA,
            pltpu.SemaphoreType.DMA,
            pltpu.SemaphoreType.DMA
        ],
        compiler_params=pltpu.CompilerParams(use_tc_tiling_on_sc=True,
                                             needs_layout_passes=False),
    )
    def body(embt_hbm, tail_hbm, out_hbm,
             vbuf0, vbuf1, vbuf2, vbuf3, wbuf0, wbuf1, wbuf2, wbuf3, tbuf,
             seml0, seml1, seml2, seml3, sems0, sems1, sems2, sems3):
        wid = lax.axis_index("s") * _NC + lax.axis_index("c")
        col0 = pl.multiple_of(wid * (_SLABS_PER_TILE * 128), 128)
        vbufs = (vbuf0, vbuf1, vbuf2, vbuf3)
        wbufs = (wbuf0, wbuf1, wbuf2, wbuf3)
        semls = (seml0, seml1, seml2, seml3)
        semss = (sems0, sems1, sems2, sems3)
        iota16 = lax.iota(jnp.int32, 16)

        def load(slab, b):
            src = embt_hbm.at[:, pl.ds(pl.multiple_of(col0 + slab * 128, 128),
                                       128)]
            pltpu.async_copy(src, vbufs[b], semls[b])

        def store(slab, b):
            dst = out_hbm.at[pl.ds(
                pl.multiple_of((col0 + slab * 128) * DIM, 2048), 2048)]
            pltpu.async_copy(wbufs[b], dst, semss[b])

        def transpose(b):
            for c in range(128):
                colidx = jnp.full((16,), c, jnp.int32)
                row = plsc.load_gather(vbufs[b], [iota16, colidx])
                wbufs[b][pl.ds(c * 16, 16)] = row

        for b in range(_NBUF):
            load(b, b)

        def ring(g, carry):
            slab0 = g * _NBUF
            for b in range(_NBUF):
                pltpu.make_async_copy(
                    embt_hbm.at[:, pl.ds(0, 128)], vbufs[b], semls[b]).wait()
                pl.when(g > 0)(lambda b=b: pltpu.make_async_copy(
                    wbufs[b], out_hbm.at[pl.ds(0, 2048)], semss[b]).wait())
                transpose(b)
                store(slab0 + b, b)
                pl.when(g < _GROUPS - 1)(
                    lambda b=b: load(slab0 + _NBUF + b, b))
            return carry

        lax.fori_loop(0, _GROUPS, ring, 0)
        for b in range(_NBUF):
            pltpu.make_async_copy(
                wbufs[b], out_hbm.at[pl.ds(0, 2048)], semss[b]).wait()

        @pl.when(wid < _EXTRA)
        def _():
            xcol = pl.multiple_of((_NW * _SLABS_PER_TILE + wid) * 128, 128)
            pltpu.async_copy(embt_hbm.at[:, pl.ds(xcol, 128)],
                             vbuf0, seml0)
            pltpu.make_async_copy(
                embt_hbm.at[:, pl.ds(0, 128)], vbuf0, seml0).wait()
            transpose(0)
            pltpu.async_copy(
                wbuf0,
                out_hbm.at[pl.ds(pl.multiple_of(xcol * DIM, 2048), 2048)],
                sems0)
            pltpu.make_async_copy(
                wbuf0, out_hbm.at[pl.ds(0, 2048)], sems0).wait()

        @pl.when(wid == _NW - 1)
        def _():
            pltpu.sync_copy(tail_hbm, tbuf)
            pltpu.sync_copy(tbuf, out_hbm.at[pl.ds(_TAIL0 * DIM, 1024)])

    return body(embt, tail)


def _sc_scores(left, right, emb_lin, bias):
    mesh = plsc.VectorSubcoreMesh(core_axis_name="c", subcore_axis_name="s")

    @functools.partial(
        pl.kernel,
        out_type=jax.ShapeDtypeStruct((128, 128), jnp.float32),
        mesh=mesh,
        scratch_types=[
            pltpu.VMEM((_NCHUNK, _CHUNK), jnp.int32),
            pltpu.VMEM((_NCHUNK, _CHUNK), jnp.int32),
            pltpu.VMEM((_BPW, DIM), jnp.float32),
            pltpu.VMEM((_BPW, DIM), jnp.float32),
            pltpu.VMEM((_BPW,), jnp.float32),
            pltpu.VMEM((_BPW // 128, 128), jnp.float32),
            pltpu.SemaphoreType.DMA,
        ],
        compiler_params=pltpu.CompilerParams(use_tc_tiling_on_sc=False,
                                             needs_layout_passes=False),
    )
    def body(left_hbm, right_hbm, emb_hbm, bias_hbm, score_hbm,
             lidx, ridx, lrows, rrows, bvals, score_v, sem):
        wid = lax.axis_index("s") * _NC + lax.axis_index("c")
        base = wid * _BPW

        for c in range(_NCHUNK):
            pltpu.sync_copy(left_hbm.at[pl.ds(base + c * _CHUNK, _CHUNK)],
                            lidx.at[c])
            pltpu.sync_copy(right_hbm.at[pl.ds(base + c * _CHUNK, _CHUNK)],
                            ridx.at[c])

        handles = []
        for c in range(_NCHUNK):
            sl = pl.ds(c * _CHUNK, _CHUNK)
            handles.append(pltpu.async_copy(emb_hbm.at[lidx.at[c]],
                                            lrows.at[sl], sem))
            handles.append(pltpu.async_copy(emb_hbm.at[ridx.at[c]],
                                            rrows.at[sl], sem))
            handles.append(pltpu.async_copy(bias_hbm.at[ridx.at[c]],
                                            bvals.at[sl], sem))
        for h in handles:
            h.wait()

        iota16 = lax.iota(jnp.int32, 16)

        def group(g, carry):
            row0 = pl.multiple_of(g * 16, 16)
            rowidx = iota16 + row0
            acc = bvals[pl.ds(row0, 16)]
            for j in range(DIM):
                colidx = jnp.full((16,), j, jnp.int32)
                lv = plsc.load_gather(lrows, [rowidx, colidx])
                rv = plsc.load_gather(rrows, [rowidx, colidx])
                acc = acc + lv * rv
            score_v[g // 8, pl.ds((g % 8) * 16, 16)] = acc
            return carry

        lax.fori_loop(0, _NGROUP, group, 0)
        pltpu.sync_copy(score_v,
                        score_hbm.at[pl.ds(wid * (_BPW // 128), _BPW // 128)])

    return body(left, right, emb_lin, bias)


def _tc_loss_kernel(score_ref, y_ref, out_ref):
    s = score_ref[...]
    y = y_ref[...]
    prob = jax.nn.sigmoid(s)
    prob = jnp.clip(prob, 1e-05, 1 - 1e-05)
    out_ref[0, 0] = -jnp.sum(y * jnp.log(prob) + (1 - y) * jnp.log(1 - prob))


def _tc_loss(score, y):
    out = pl.pallas_call(
        _tc_loss_kernel,
        out_shape=jax.ShapeDtypeStruct((1, 1), jnp.float32),
        out_specs=pl.BlockSpec(memory_space=pltpu.SMEM),
    )(score, y.reshape(128, 128))
    return out[0, 0]


def kernel(left, right, y, emb, bias):
    tail = emb[_TAIL0:, :].reshape(1024)
    flat = _sc_convert(emb.T, tail)
    emb_lin = flat.reshape(N, DIM)
    score = _sc_scores(left.astype(jnp.int32), right.astype(jnp.int32),
                       emb_lin, bias)
    return _tc_loss(score, y)


# speedup vs baseline: 10.7369x; 3.9066x over previous
import functools

import jax
import jax.numpy as jnp
from jax import lax
from jax.experimental import pallas as pl
from jax.experimental.pallas import tpu as pltpu
from jax.experimental.pallas import tpu_sc as plsc

N = 1000000
DIM = 16
B = 16384

_NC = 2
_NS = 16
_NW = _NC * _NS

_TCOLS = N // 128
_SLABS_PER_TILE = _TCOLS // _NW
_EXTRA = _TCOLS - _SLABS_PER_TILE * _NW
_TAIL0 = _TCOLS * 128
_FLAT = (_TCOLS + 1) * 2048
_NBUF = 4
_GROUPS = _SLABS_PER_TILE // _NBUF

_BPW = B // _NW
_CHUNK = 128
_NCHUNK = _BPW // _CHUNK
_NGROUP = _BPW // 16


def _sc_convert(embt, tail):
    mesh = plsc.VectorSubcoreMesh(core_axis_name="c", subcore_axis_name="s")

    @functools.partial(
        pl.kernel,
        out_type=jax.ShapeDtypeStruct((_FLAT,), jnp.float32),
        mesh=mesh,
        scratch_types=[
            pltpu.VMEM((DIM, 128), jnp.float32),
            pltpu.VMEM((DIM, 128), jnp.float32),
            pltpu.VMEM((DIM, 128), jnp.float32),
            pltpu.VMEM((DIM, 128), jnp.float32),
            pltpu.VMEM((1024,), jnp.float32),
            pltpu.SemaphoreType.DMA,
            pltpu.SemaphoreType.DMA,
            pltpu.SemaphoreType.DMA,
            pltpu.SemaphoreType.DMA,
            pltpu.SemaphoreType.DMA,
            pltpu.SemaphoreType.DMA,
            pltpu.SemaphoreType.DMA,
            pltpu.SemaphoreType.DMA,
        ],
        compiler_params=pltpu.CompilerParams(use_tc_tiling_on_sc=True),
    )
    def body(embt_hbm, tail_hbm, out_hbm,
             vbuf0, vbuf1, vbuf2, vbuf3, tbuf,
             seml0, seml1, seml2, seml3, sems0, sems1, sems2, sems3):
        wid = lax.axis_index("s") * _NC + lax.axis_index("c")
        slab0w = wid * _SLABS_PER_TILE
        vbufs = (vbuf0, vbuf1, vbuf2, vbuf3)
        semls = (seml0, seml1, seml2, seml3)
        semss = (sems0, sems1, sems2, sems3)

        def load(slab, b):
            src = embt_hbm.at[:, pl.ds(
                pl.multiple_of((slab0w + slab) * 128, 128), 128)]
            pltpu.async_copy(src, vbufs[b], semls[b])

        def store(slab, b):
            base = pl.multiple_of((slab0w + slab) * 2048, 2048)
            for j in range(DIM):
                pltpu.async_copy(vbufs[b].at[j],
                                 out_hbm.at[pl.ds(base + j * 128, 128)],
                                 semss[b])

        def wait_load(b):
            pltpu.make_async_copy(
                embt_hbm.at[:, pl.ds(0, 128)], vbufs[b], semls[b]).wait()

        def wait_store(b):
            for _ in range(DIM):
                pltpu.make_async_copy(
                    vbufs[b].at[0], out_hbm.at[pl.ds(0, 128)],
                    semss[b]).wait()

        for b in range(_NBUF):
            load(b, b)

        def ring(g, carry):
            slab0 = g * _NBUF
            for b in range(_NBUF):
                wait_load(b)
                pl.when(g > 0)(lambda b=b: wait_store(b))
                store(slab0 + b, b)
                pl.when(g < _GROUPS - 1)(
                    lambda b=b, s=0: load(slab0 + _NBUF + b, b))
            return carry

        lax.fori_loop(0, _GROUPS, ring, 0)
        for b in range(_NBUF):
            wait_store(b)

        @pl.when(wid < _EXTRA)
        def _():
            xslab = _NW * _SLABS_PER_TILE + wid
            src = embt_hbm.at[:, pl.ds(pl.multiple_of(xslab * 128, 128), 128)]
            pltpu.async_copy(src, vbuf0, seml0)
            wait_load(0)
            base = pl.multiple_of(xslab * 2048, 2048)
            for j in range(DIM):
                pltpu.async_copy(vbuf0.at[j],
                                 out_hbm.at[pl.ds(base + j * 128, 128)],
                                 sems0)
            wait_store(0)

        @pl.when(wid == _NW - 1)
        def _():
            pltpu.sync_copy(tail_hbm, tbuf)
            for j in range(DIM):
                pltpu.sync_copy(
                    tbuf.at[pl.ds(j * 64, 64)],
                    out_hbm.at[pl.ds(_TCOLS * 2048 + j * 128, 64)])

    return body(embt, tail)


def _sc_scores(left, right, flat, bias):
    mesh = plsc.VectorSubcoreMesh(core_axis_name="c", subcore_axis_name="s")

    @functools.partial(
        pl.kernel,
        out_type=jax.ShapeDtypeStruct((128, 128), jnp.float32),
        mesh=mesh,
        scratch_types=[
            pltpu.VMEM((_NCHUNK, _CHUNK), jnp.int32),
            pltpu.VMEM((_NCHUNK, _CHUNK), jnp.int32),
            pltpu.VMEM((_NCHUNK, _CHUNK), jnp.int32),
            pltpu.VMEM((DIM, _BPW), jnp.float32),
            pltpu.VMEM((DIM, _BPW), jnp.float32),
            pltpu.VMEM((_BPW,), jnp.float32),
            pltpu.VMEM((_BPW // 128, 128), jnp.float32),
            pltpu.SemaphoreType.DMA,
        ],
        compiler_params=pltpu.CompilerParams(use_tc_tiling_on_sc=False,
                                             needs_layout_passes=False),
    )
    def body(left_hbm, right_hbm, flat_hbm, bias_hbm, score_hbm,
             ridx, tlidx, tridx, lcols, rcols, bvals, score_v, sem):
        wid = lax.axis_index("s") * _NC + lax.axis_index("c")
        base = wid * _BPW

        for c in range(_NCHUNK):
            pltpu.sync_copy(left_hbm.at[pl.ds(base + c * _CHUNK, _CHUNK)],
                            tlidx.at[c])
            pltpu.sync_copy(right_hbm.at[pl.ds(base + c * _CHUNK, _CHUNK)],
                            ridx.at[c])

        for c in range(_NCHUNK):
            for q in range(_CHUNK // 16):
                sl = pl.ds(q * 16, 16)
                iv = tlidx[c, sl]
                tlidx[c, sl] = iv + (iv >> 7) * 1920
                rv = ridx[c, sl]
                tridx[c, sl] = rv + (rv >> 7) * 1920

        handles = []
        for c in range(_NCHUNK):
            sl = pl.ds(c * _CHUNK, _CHUNK)
            handles.append(pltpu.async_copy(bias_hbm.at[ridx.at[c]],
                                            bvals.at[sl], sem))
            for j in range(DIM):
                view = flat_hbm.at[pl.ds(j * 128, _FLAT - 128 * j)]
                handles.append(pltpu.async_copy(
                    view.at[tlidx.at[c]], lcols.at[j, sl], sem))
                handles.append(pltpu.async_copy(
                    view.at[tridx.at[c]], rcols.at[j, sl], sem))
        for h in handles:
            h.wait()

        for g in range(_NGROUP):
            sl = pl.ds(g * 16, 16)
            acc = bvals[sl]
            for j in range(DIM):
                acc = acc + lcols[j, sl] * rcols[j, sl]
            score_v[g // 8, pl.ds((g % 8) * 16, 16)] = acc

        pltpu.sync_copy(score_v,
                        score_hbm.at[pl.ds(wid * (_BPW // 128), _BPW // 128)])

    return body(left, right, flat, bias)


def _tc_loss_kernel(score_ref, y_ref, out_ref):
    s = score_ref[...]
    y = y_ref[...]
    prob = jax.nn.sigmoid(s)
    prob = jnp.clip(prob, 1e-05, 1 - 1e-05)
    out_ref[0, 0] = -jnp.sum(y * jnp.log(prob) + (1 - y) * jnp.log(1 - prob))


def _tc_loss(score, y):
    out = pl.pallas_call(
        _tc_loss_kernel,
        out_shape=jax.ShapeDtypeStruct((1, 1), jnp.float32),
        out_specs=pl.BlockSpec(memory_space=pltpu.SMEM),
    )(score, y.reshape(128, 128))
    return out[0, 0]


def kernel(left, right, y, emb, bias):
    tail = emb[_TAIL0:, :].T.reshape(1024)
    flat = _sc_convert(emb.T, tail)
    score = _sc_scores(left.astype(jnp.int32), right.astype(jnp.int32),
                       flat, bias)
    return _tc_loss(score, y)


# speedup vs baseline: 12.6794x; 1.1809x over previous
import functools

import jax
import jax.numpy as jnp
from jax import lax
from jax.experimental import pallas as pl
from jax.experimental.pallas import tpu as pltpu
from jax.experimental.pallas import tpu_sc as plsc

N = 1000000
DIM = 16
B = 16384

_NC = 2
_NS = 16
_NW = _NC * _NS

_TCOLS = N // 128
_SLABS_PER_TILE = _TCOLS // _NW
_EXTRA = _TCOLS - _SLABS_PER_TILE * _NW
_TAIL0 = _TCOLS * 128
_NPAIR = DIM // 2
_WSLAB = _NPAIR * 128
_FLATW = (_TCOLS + 1) * _WSLAB
_NBUF = 4
_GROUPS = _SLABS_PER_TILE // _NBUF

_BPW = B // _NW
_CHUNK = 128
_NCHUNK = _BPW // _CHUNK
_NGROUP = _BPW // 16


def _sc_convert(embt, tailw):
    mesh = plsc.VectorSubcoreMesh(core_axis_name="c", subcore_axis_name="s")

    @functools.partial(
        pl.kernel,
        out_type=jax.ShapeDtypeStruct((_FLATW,), jnp.int32),
        mesh=mesh,
        scratch_types=[
            pltpu.VMEM((DIM, 128), jnp.float32),
            pltpu.VMEM((DIM, 128), jnp.float32),
            pltpu.VMEM((DIM, 128), jnp.float32),
            pltpu.VMEM((DIM, 128), jnp.float32),
            pltpu.VMEM((_WSLAB,), jnp.int32),
            pltpu.VMEM((_WSLAB,), jnp.int32),
            pltpu.VMEM((_WSLAB,), jnp.int32),
            pltpu.VMEM((_WSLAB,), jnp.int32),
            pltpu.VMEM((_WSLAB // 2,), jnp.int32),
            pltpu.SemaphoreType.DMA,
            pltpu.SemaphoreType.DMA,
            pltpu.SemaphoreType.DMA,
            pltpu.SemaphoreType.DMA,
            pltpu.SemaphoreType.DMA,
            pltpu.SemaphoreType.DMA,
            pltpu.SemaphoreType.DMA,
            pltpu.SemaphoreType.DMA,
        ],
        compiler_params=pltpu.CompilerParams(use_tc_tiling_on_sc=True,
                                             needs_layout_passes=False),
    )
    def body(embt_hbm, tail_hbm, out_hbm,
             vbuf0, vbuf1, vbuf2, vbuf3, wbuf0, wbuf1, wbuf2, wbuf3, tbuf,
             seml0, seml1, seml2, seml3, sems0, sems1, sems2, sems3):
        wid = lax.axis_index("s") * _NC + lax.axis_index("c")
        slab0w = wid * _SLABS_PER_TILE
        vbufs = (vbuf0, vbuf1, vbuf2, vbuf3)
        wbufs = (wbuf0, wbuf1, wbuf2, wbuf3)
        semls = (seml0, seml1, seml2, seml3)
        semss = (sems0, sems1, sems2, sems3)

        def load(slab, b):
            src = embt_hbm.at[:, pl.ds(
                pl.multiple_of((slab0w + slab) * 128, 128), 128)]
            pltpu.async_copy(src, vbufs[b], semls[b])

        def pack_slab(b):
            for p in range(_NPAIR):
                for q in range(8):
                    sl = pl.ds(q * 16, 16)
                    a = vbufs[b][2 * p, sl]
                    c = vbufs[b][2 * p + 1, sl]
                    w = plsc.bitcast(
                        plsc.pack(a, c, format=plsc.PackFormat.INTERLEAVED),
                        jnp.int32)
                    wbufs[b][pl.ds(p * 128 + q * 16, 16)] = w

        def store(slab, b):
            base = pl.multiple_of((slab0w + slab) * _WSLAB, _WSLAB)
            pltpu.async_copy(wbufs[b], out_hbm.at[pl.ds(base, _WSLAB)],
                             semss[b])

        def wait_load(b):
            pltpu.make_async_copy(
                embt_hbm.at[:, pl.ds(0, 128)], vbufs[b], semls[b]).wait()

        def wait_store(b):
            pltpu.make_async_copy(
                wbufs[b], out_hbm.at[pl.ds(0, _WSLAB)], semss[b]).wait()

        for b in range(_NBUF):
            load(b, b)

        def ring(g, carry):
            slab0 = g * _NBUF
            for b in range(_NBUF):
                wait_load(b)
                pl.when(g > 0)(lambda b=b: wait_store(b))
                pack_slab(b)
                store(slab0 + b, b)
                pl.when(g < _GROUPS - 1)(
                    lambda b=b: load(slab0 + _NBUF + b, b))
            return carry

        lax.fori_loop(0, _GROUPS, ring, 0)
        for b in range(_NBUF):
            wait_store(b)

        @pl.when(wid < _EXTRA)
        def _():
            xslab = _NW * _SLABS_PER_TILE + wid
            src = embt_hbm.at[:, pl.ds(pl.multiple_of(xslab * 128, 128), 128)]
            pltpu.async_copy(src, vbuf0, seml0)
            wait_load(0)
            pack_slab(0)
            pltpu.async_copy(
                wbuf0,
                out_hbm.at[pl.ds(pl.multiple_of(xslab * _WSLAB, _WSLAB),
                                 _WSLAB)],
                sems0)
            wait_store(0)

        @pl.when(wid == _NW - 1)
        def _():
            pltpu.sync_copy(tail_hbm, tbuf)
            for p in range(_NPAIR):
                pltpu.sync_copy(
                    tbuf.at[pl.ds(p * 64, 64)],
                    out_hbm.at[pl.ds(_TCOLS * _WSLAB + p * 128, 64)])

    return body(embt, tailw)


def _sc_scores(left, right, flatw, bias):
    mesh = plsc.VectorSubcoreMesh(core_axis_name="c", subcore_axis_name="s")

    @functools.partial(
        pl.kernel,
        out_type=jax.ShapeDtypeStruct((128, 128), jnp.float32),
        mesh=mesh,
        scratch_types=[
            pltpu.VMEM((_NCHUNK, _CHUNK), jnp.int32),
            pltpu.VMEM((_NCHUNK, _CHUNK), jnp.int32),
            pltpu.VMEM((_NCHUNK, _CHUNK), jnp.int32),
            pltpu.VMEM((_NPAIR, _BPW), jnp.int32),
            pltpu.VMEM((_NPAIR, _BPW), jnp.int32),
            pltpu.VMEM((_BPW,), jnp.float32),
            pltpu.VMEM((_BPW // 128, 128), jnp.float32),
            pltpu.SemaphoreType.DMA,
        ],
        compiler_params=pltpu.CompilerParams(use_tc_tiling_on_sc=False,
                                             needs_layout_passes=False),
    )
    def body(left_hbm, right_hbm, flatw_hbm, bias_hbm, score_hbm,
             ridx, tlidx, tridx, lcols, rcols, bvals, score_v, sem):
        wid = lax.axis_index("s") * _NC + lax.axis_index("c")
        base = wid * _BPW

        for c in range(_NCHUNK):
            pltpu.sync_copy(left_hbm.at[pl.ds(base + c * _CHUNK, _CHUNK)],
                            tlidx.at[c])
            pltpu.sync_copy(right_hbm.at[pl.ds(base + c * _CHUNK, _CHUNK)],
                            ridx.at[c])

        for c in range(_NCHUNK):
            for q in range(_CHUNK // 16):
                sl = pl.ds(q * 16, 16)
                iv = tlidx[c, sl]
                tlidx[c, sl] = iv + (iv >> 7) * 896
                rv = ridx[c, sl]
                tridx[c, sl] = rv + (rv >> 7) * 896

        handles = []
        for c in range(_NCHUNK):
            sl = pl.ds(c * _CHUNK, _CHUNK)
            handles.append(pltpu.async_copy(bias_hbm.at[ridx.at[c]],
                                            bvals.at[sl], sem))
            for p in range(_NPAIR):
                view = flatw_hbm.at[pl.ds(p * 128, _FLATW - 128 * p)]
                handles.append(pltpu.async_copy(
                    view.at[tlidx.at[c]], lcols.at[p, sl], sem))
                handles.append(pltpu.async_copy(
                    view.at[tridx.at[c]], rcols.at[p, sl], sem))
        for h in handles:
            h.wait()

        for g in range(_NGROUP):
            sl = pl.ds(g * 16, 16)
            acc = bvals[sl]
            for p in range(_NPAIR):
                la, lb = plsc.unpack(
                    plsc.bitcast(lcols[p, sl], jnp.bfloat16),
                    format=plsc.PackFormat.INTERLEAVED)
                ra, rb = plsc.unpack(
                    plsc.bitcast(rcols[p, sl], jnp.bfloat16),
                    format=plsc.PackFormat.INTERLEAVED)
                acc = acc + la * ra + lb * rb
            score_v[g // 8, pl.ds((g % 8) * 16, 16)] = acc

        pltpu.sync_copy(score_v,
                        score_hbm.at[pl.ds(wid * (_BPW // 128), _BPW // 128)])

    return body(left, right, flatw, bias)


def _tc_loss_kernel(score_ref, y_ref, out_ref):
    s = score_ref[...]
    y = y_ref[...]
    prob = jax.nn.sigmoid(s)
    prob = jnp.clip(prob, 1e-05, 1 - 1e-05)
    out_ref[0, 0] = -jnp.sum(y * jnp.log(prob) + (1 - y) * jnp.log(1 - prob))


def _tc_loss(score, y):
    out = pl.pallas_call(
        _tc_loss_kernel,
        out_shape=jax.ShapeDtypeStruct((1, 1), jnp.float32),
        out_specs=pl.BlockSpec(memory_space=pltpu.SMEM),
    )(score, y.reshape(128, 128))
    return out[0, 0]


def kernel(left, right, y, emb, bias):
    tailt = emb[_TAIL0:, :].T
    pk = jnp.stack([tailt[0::2].astype(jnp.bfloat16),
                    tailt[1::2].astype(jnp.bfloat16)], axis=-1)
    tailw = jax.lax.bitcast_convert_type(pk, jnp.int32).reshape(512)
    flatw = _sc_convert(emb.T, tailw)
    score = _sc_scores(left.astype(jnp.int32), right.astype(jnp.int32),
                       flatw, bias)
    return _tc_loss(score, y)


# speedup vs baseline: 13.3754x; 1.0549x over previous
import functools

import jax
import jax.numpy as jnp
from jax import lax
from jax.experimental import pallas as pl
from jax.experimental.pallas import tpu as pltpu
from jax.experimental.pallas import tpu_sc as plsc

N = 1000000
DIM = 16
B = 16384

_NC = 2
_NS = 16
_NW = _NC * _NS

_TCOLS = N // 128
_SLABS_PER_TILE = _TCOLS // _NW
_EXTRA = _TCOLS - _SLABS_PER_TILE * _NW
_TAIL0 = _TCOLS * 128
_NQUAD = DIM // 4
_WSLAB = _NQUAD * 128
_FLATW = (_TCOLS + 1) * _WSLAB
_NBUF = 4
_GROUPS = _SLABS_PER_TILE // _NBUF

_BPW = B // _NW
_CHUNK = 128
_NCHUNK = _BPW // _CHUNK
_NGROUP = _BPW // 16


def _sc_convert(embt, tailw):
    mesh = plsc.VectorSubcoreMesh(core_axis_name="c", subcore_axis_name="s")

    @functools.partial(
        pl.kernel,
        out_type=jax.ShapeDtypeStruct((_FLATW,), jnp.int32),
        mesh=mesh,
        scratch_types=[
            pltpu.VMEM((DIM, 128), jnp.float32),
            pltpu.VMEM((DIM, 128), jnp.float32),
            pltpu.VMEM((DIM, 128), jnp.float32),
            pltpu.VMEM((DIM, 128), jnp.float32),
            pltpu.VMEM((_WSLAB,), jnp.int32),
            pltpu.VMEM((_WSLAB,), jnp.int32),
            pltpu.VMEM((_WSLAB,), jnp.int32),
            pltpu.VMEM((_WSLAB,), jnp.int32),
            pltpu.VMEM((_WSLAB // 2,), jnp.int32),
            pltpu.SemaphoreType.DMA,
            pltpu.SemaphoreType.DMA,
            pltpu.SemaphoreType.DMA,
            pltpu.SemaphoreType.DMA,
            pltpu.SemaphoreType.DMA,
            pltpu.SemaphoreType.DMA,
            pltpu.SemaphoreType.DMA,
            pltpu.SemaphoreType.DMA,
        ],
        compiler_params=pltpu.CompilerParams(use_tc_tiling_on_sc=True,
                                             needs_layout_passes=False),
    )
    def body(embt_hbm, tail_hbm, out_hbm,
             vbuf0, vbuf1, vbuf2, vbuf3, wbuf0, wbuf1, wbuf2, wbuf3, tbuf,
             seml0, seml1, seml2, seml3, sems0, sems1, sems2, sems3):
        wid = lax.axis_index("s") * _NC + lax.axis_index("c")
        slab0w = wid * _SLABS_PER_TILE
        vbufs = (vbuf0, vbuf1, vbuf2, vbuf3)
        wbufs = (wbuf0, wbuf1, wbuf2, wbuf3)
        semls = (seml0, seml1, seml2, seml3)
        semss = (sems0, sems1, sems2, sems3)

        def load(slab, b):
            src = embt_hbm.at[:, pl.ds(
                pl.multiple_of((slab0w + slab) * 128, 128), 128)]
            pltpu.async_copy(src, vbufs[b], semls[b])

        def pack_slab(b):
            for p in range(_NQUAD):
                for q in range(8):
                    sl = pl.ds(q * 16, 16)
                    x = plsc.pack(vbufs[b][4 * p, sl], vbufs[b][4 * p + 1, sl],
                                  format=plsc.PackFormat.INTERLEAVED)
                    z = plsc.pack(vbufs[b][4 * p + 2, sl],
                                  vbufs[b][4 * p + 3, sl],
                                  format=plsc.PackFormat.INTERLEAVED)
                    w = plsc.bitcast(
                        plsc.pack(x, z, format=plsc.PackFormat.INTERLEAVED,
                                  preferred_element_type=jnp.float8_e4m3fn),
                        jnp.int32)
                    wbufs[b][pl.ds(p * 128 + q * 16, 16)] = w

        def store(slab, b):
            base = pl.multiple_of((slab0w + slab) * _WSLAB, _WSLAB)
            pltpu.async_copy(wbufs[b], out_hbm.at[pl.ds(base, _WSLAB)],
                             semss[b])

        def wait_load(b):
            pltpu.make_async_copy(
                embt_hbm.at[:, pl.ds(0, 128)], vbufs[b], semls[b]).wait()

        def wait_store(b):
            pltpu.make_async_copy(
                wbufs[b], out_hbm.at[pl.ds(0, _WSLAB)], semss[b]).wait()

        for b in range(_NBUF):
            load(b, b)

        def ring(g, carry):
            slab0 = g * _NBUF
            for b in range(_NBUF):
                wait_load(b)
                pl.when(g > 0)(lambda b=b: wait_store(b))
                pack_slab(b)
                store(slab0 + b, b)
                pl.when(g < _GROUPS - 1)(
                    lambda b=b: load(slab0 + _NBUF + b, b))
            return carry

        lax.fori_loop(0, _GROUPS, ring, 0)
        for b in range(_NBUF):
            wait_store(b)

        @pl.when(wid < _EXTRA)
        def _():
            xslab = _NW * _SLABS_PER_TILE + wid
            src = embt_hbm.at[:, pl.ds(pl.multiple_of(xslab * 128, 128), 128)]
            pltpu.async_copy(src, vbuf0, seml0)
            wait_load(0)
            pack_slab(0)
            pltpu.async_copy(
                wbuf0,
                out_hbm.at[pl.ds(pl.multiple_of(xslab * _WSLAB, _WSLAB),
                                 _WSLAB)],
                sems0)
            wait_store(0)

        @pl.when(wid == _NW - 1)
        def _():
            pltpu.sync_copy(tail_hbm, tbuf)
            for p in range(_NQUAD):
                pltpu.sync_copy(
                    tbuf.at[pl.ds(p * 64, 64)],
                    out_hbm.at[pl.ds(_TCOLS * _WSLAB + p * 128, 64)])

    return body(embt, tailw)


def _sc_scores(left, right, flatw, bias):
    mesh = plsc.VectorSubcoreMesh(core_axis_name="c", subcore_axis_name="s")

    @functools.partial(
        pl.kernel,
        out_type=jax.ShapeDtypeStruct((128, 128), jnp.float32),
        mesh=mesh,
        scratch_types=[
            pltpu.VMEM((_NCHUNK, _CHUNK), jnp.int32),
            pltpu.VMEM((_NCHUNK, _CHUNK), jnp.int32),
            pltpu.VMEM((_NCHUNK, _CHUNK), jnp.int32),
            pltpu.VMEM((_NQUAD, _BPW), jnp.int32),
            pltpu.VMEM((_NQUAD, _BPW), jnp.int32),
            pltpu.VMEM((_BPW,), jnp.float32),
            pltpu.VMEM((_BPW // 128, 128), jnp.float32),
            pltpu.SemaphoreType.DMA,
        ],
        compiler_params=pltpu.CompilerParams(use_tc_tiling_on_sc=False,
                                             needs_layout_passes=False),
    )
    def body(left_hbm, right_hbm, flatw_hbm, bias_hbm, score_hbm,
             ridx, tlidx, tridx, lcols, rcols, bvals, score_v, sem):
        wid = lax.axis_index("s") * _NC + lax.axis_index("c")
        base = wid * _BPW

        for c in range(_NCHUNK):
            pltpu.sync_copy(left_hbm.at[pl.ds(base + c * _CHUNK, _CHUNK)],
                            tlidx.at[c])
            pltpu.sync_copy(right_hbm.at[pl.ds(base + c * _CHUNK, _CHUNK)],
                            ridx.at[c])

        for c in range(_NCHUNK):
            for q in range(_CHUNK // 16):
                sl = pl.ds(q * 16, 16)
                iv = tlidx[c, sl]
                tlidx[c, sl] = iv + (iv >> 7) * 384
                rv = ridx[c, sl]
                tridx[c, sl] = rv + (rv >> 7) * 384

        handles = []
        for c in range(_NCHUNK):
            sl = pl.ds(c * _CHUNK, _CHUNK)
            handles.append(pltpu.async_copy(bias_hbm.at[ridx.at[c]],
                                            bvals.at[sl], sem))
            for p in range(_NQUAD):
                view = flatw_hbm.at[pl.ds(p * 128, _FLATW - 128 * p)]
                handles.append(pltpu.async_copy(
                    view.at[tlidx.at[c]], lcols.at[p, sl], sem))
                handles.append(pltpu.async_copy(
                    view.at[tridx.at[c]], rcols.at[p, sl], sem))
        for h in handles:
            h.wait()

        for g in range(_NGROUP):
            sl = pl.ds(g * 16, 16)
            acc = bvals[sl]
            for p in range(_NQUAD):
                lx, lz = plsc.unpack(
                    plsc.bitcast(lcols[p, sl], jnp.float8_e4m3fn),
                    format=plsc.PackFormat.INTERLEAVED,
                    preferred_element_type=jnp.bfloat16)
                rx, rz = plsc.unpack(
                    plsc.bitcast(rcols[p, sl], jnp.float8_e4m3fn),
                    format=plsc.PackFormat.INTERLEAVED,
                    preferred_element_type=jnp.bfloat16)
                la, lb = plsc.unpack(lx, format=plsc.PackFormat.INTERLEAVED)
                lc, ld = plsc.unpack(lz, format=plsc.PackFormat.INTERLEAVED)
                ra, rb = plsc.unpack(rx, format=plsc.PackFormat.INTERLEAVED)
                rc, rd = plsc.unpack(rz, format=plsc.PackFormat.INTERLEAVED)
                acc = acc + la * ra + lb * rb + lc * rc + ld * rd
            score_v[g // 8, pl.ds((g % 8) * 16, 16)] = acc

        pltpu.sync_copy(score_v,
                        score_hbm.at[pl.ds(wid * (_BPW // 128), _BPW // 128)])

    return body(left, right, flatw, bias)


def _tc_loss_kernel(score_ref, y_ref, out_ref):
    s = score_ref[...]
    y = y_ref[...]
    prob = jax.nn.sigmoid(s)
    prob = jnp.clip(prob, 1e-05, 1 - 1e-05)
    out_ref[0, 0] = -jnp.sum(y * jnp.log(prob) + (1 - y) * jnp.log(1 - prob))


def _tc_loss(score, y):
    out = pl.pallas_call(
        _tc_loss_kernel,
        out_shape=jax.ShapeDtypeStruct((1, 1), jnp.float32),
        out_specs=pl.BlockSpec(memory_space=pltpu.SMEM),
    )(score, y.reshape(128, 128))
    return out[0, 0]


def kernel(left, right, y, emb, bias):
    tailt = emb[_TAIL0:, :].T
    f8 = jnp.float8_e4m3fn
    pk = jnp.stack([tailt[0::4].astype(f8), tailt[2::4].astype(f8),
                    tailt[1::4].astype(f8), tailt[3::4].astype(f8)],
                   axis=-1)
    tailw = jax.lax.bitcast_convert_type(pk, jnp.int32).reshape(256)
    flatw = _sc_convert(emb.T, tailw)
    score = _sc_scores(left.astype(jnp.int32), right.astype(jnp.int32),
                       flatw, bias)
    return _tc_loss(score, y)
